# Initial kernel scaffold; baseline (speedup 1.0000x reference)
import jax
import jax.numpy as jnp
from jax.experimental import pallas as pl


def kernel(dense_features, categorical_features, tables, W0, b0, W1, b1, Wt0, bt0, Wt1, bt1):
    raise NotImplementedError("write your pallas kernel here")


# speedup vs baseline: 8.0335x; 8.0335x over previous
import functools

import jax
import jax.numpy as jnp
from jax import lax
from jax.experimental import pallas as pl
from jax.experimental.pallas import tpu as pltpu
from jax.experimental.pallas import tpu_sc as plsc

B = 16384
D = 32
F = 26
V = 100000
NC, NS = 2, 16
NW = NC * NS
BW = B // NW
NCHUNK = 4
CH = BW // NCHUNK


def _sc_gather(table_flat, gidx):
    mesh = plsc.VectorSubcoreMesh(core_axis_name="c", subcore_axis_name="s")

    @functools.partial(
        pl.kernel,
        out_type=jax.ShapeDtypeStruct((B, F * D), jnp.float32),
        mesh=mesh,
        compiler_params=pltpu.CompilerParams(use_tc_tiling_on_sc=False),
        scratch_types=[
            pltpu.VMEM((F, NCHUNK, CH), jnp.int32),
            pltpu.VMEM((BW, D), jnp.float32),
            pltpu.SemaphoreType.DMA,
            pltpu.SemaphoreType.DMA,
            pltpu.SemaphoreType.DMA,
            pltpu.SemaphoreType.DMA,
        ],
    )
    def k(tab_hbm, idx_hbm, out_hbm, idx_v, rows_v, s0, s1, s2, s3):
        wid = lax.axis_index("s") * NC + lax.axis_index("c")
        base = wid * BW
        pltpu.sync_copy(idx_hbm.at[:, wid], idx_v)
        sems = [s0, s1, s2, s3]

        @pl.loop(0, F)
        def _(f):
            copies = []
            for c in range(NCHUNK):
                copies.append(pltpu.async_copy(
                    tab_hbm.at[idx_v.at[f, c]],
                    rows_v.at[pl.ds(c * CH, CH)],
                    sems[c],
                ))
            for cp in copies:
                cp.wait()
            pltpu.sync_copy(
                rows_v, out_hbm.at[pl.ds(base, BW), pl.ds(f * D, D)])

    return k(table_flat, gidx)


def _tc_bottom(dense, W0, b0, W1, b1):
    T = 2048

    def body(x_ref, w0_ref, b0_ref, w1_ref, b1_ref, o_ref):
        h = jnp.dot(x_ref[...], w0_ref[...],
                    preferred_element_type=jnp.float32) + b0_ref[...]
        h = jnp.maximum(h, 0.0)
        o_ref[...] = jnp.dot(h, w1_ref[...],
                             preferred_element_type=jnp.float32) + b1_ref[...]

    return pl.pallas_call(
        body,
        grid=(B // T,),
        in_specs=[
            pl.BlockSpec((T, 13), lambda i: (i, 0)),
            pl.BlockSpec((13, 256), lambda i: (0, 0)),
            pl.BlockSpec((1, 256), lambda i: (0, 0)),
            pl.BlockSpec((256, 32), lambda i: (0, 0)),
            pl.BlockSpec((1, 32), lambda i: (0, 0)),
        ],
        out_specs=pl.BlockSpec((T, 32), lambda i: (i, 0)),
        out_shape=jax.ShapeDtypeStruct((B, 32), jnp.float32),
    )(dense, W0, b0.reshape(1, 256), W1, b1.reshape(1, 32))


def _tc_top(dense_out, emb, Wt0a, Wt0b, bt0, Wt1, bt1):
    T = 1024

    def body(d_ref, e_ref, wa_ref, wb_ref, c0_ref, w1_ref, c1_ref, o_ref):
        acc = jnp.dot(d_ref[...], wa_ref[...],
                      preferred_element_type=jnp.float32)
        acc = acc + jnp.dot(e_ref[...], wb_ref[...],
                            preferred_element_type=jnp.float32)
        h2 = jnp.maximum(acc + c0_ref[...], 0.0)
        o = jnp.dot(h2, w1_ref[...],
                    preferred_element_type=jnp.float32) + c1_ref[...]
        o_ref[...] = jax.nn.sigmoid(o) * 5.0

    return pl.pallas_call(
        body,
        grid=(B // T,),
        in_specs=[
            pl.BlockSpec((T, 32), lambda i: (i, 0)),
            pl.BlockSpec((T, F * D), lambda i: (i, 0)),
            pl.BlockSpec((32, 64), lambda i: (0, 0)),
            pl.BlockSpec((F * D, 64), lambda i: (0, 0)),
            pl.BlockSpec((1, 64), lambda i: (0, 0)),
            pl.BlockSpec((64, 1), lambda i: (0, 0)),
            pl.BlockSpec((1, 1), lambda i: (0, 0)),
        ],
        out_specs=pl.BlockSpec((T, 1), lambda i: (i, 0)),
        out_shape=jax.ShapeDtypeStruct((B, 1), jnp.float32),
    )(dense_out, emb, Wt0a, Wt0b, bt0.reshape(1, 64), Wt1,
      bt1.reshape(1, 1))


def kernel(dense_features, categorical_features, tables,
           W0, b0, W1, b1, Wt0, bt0, Wt1, bt1):
    table_flat = tables.reshape(F * V, D)
    offs = (jnp.arange(F, dtype=jnp.int32) * V)[:, None]
    gidx = (categorical_features.astype(jnp.int32) + offs).reshape(
        F, NW, NCHUNK, CH)
    emb = _sc_gather(table_flat, gidx)
    dense_out = _tc_bottom(dense_features, W0, b0, W1, b1)
    out = _tc_top(dense_out, emb, Wt0[:32], Wt0[32:], bt0, Wt1, bt1)
    return out.reshape(B)
